# baseline (speedup 1.0000x reference)
import functools

import jax
import jax.numpy as jnp
from jax.experimental import pallas as pl
from jax.experimental.pallas import tpu as pltpu

_M = 1000000
_D = 64
_BR = 4000


def _fused_body(idx_ref, e2_ref, h_ref, went_ref, bent_ref, wdelta_ref,
                bdelta_ref, eout_ref, lg_ref, loss_ref, m_ref, s_ref, t_ref):
    step = pl.program_id(0)
    nsteps = pl.num_programs(0)
    idx = idx_ref[0]

    h2 = h_ref[...]
    proj = jax.lax.dot_general(h2, went_ref[...], (((1,), (0,)), ((), ())),
                               preferred_element_type=jnp.float32)
    pcat = jnp.concatenate([proj, proj], axis=1)
    hb = jnp.sum(h2 * bent_ref[...])

    blk = e2_ref[...]
    z = blk * pcat
    lane = jax.lax.broadcasted_iota(jnp.int32, (128, 2), 0)
    col = jax.lax.broadcasted_iota(jnp.int32, (128, 2), 1)
    sel = jnp.where(col == 0, (lane < 64).astype(jnp.float32),
                    (lane >= 64).astype(jnp.float32))
    lgp = jax.lax.dot_general(z, sel, (((1,), (0,)), ((), ())),
                              preferred_element_type=jnp.float32) + hb
    lg_ref[...] = lgp

    @pl.when(step == 0)
    def _init():
        m_ref[...] = jnp.full((1, 128), -1e30, jnp.float32)
        s_ref[...] = jnp.zeros((1, 128), jnp.float32)
        t_ref[...] = jnp.zeros((1, 128), jnp.float32)

    bm = jnp.max(lgp)
    bs = jnp.sum(jnp.exp(lgp - bm))
    bm_v = jnp.full((1, 128), bm, jnp.float32)
    bs_v = jnp.full((1, 128), bs, jnp.float32)
    m_old = m_ref[...]
    m_new = jnp.maximum(m_old, bm_v)
    s_ref[...] = s_ref[...] * jnp.exp(m_old - m_new) + bs_v * jnp.exp(bm_v - m_new)
    m_ref[...] = m_new

    eout_ref[...] = blk

    trow = idx // 2
    tstep = trow // _BR

    @pl.when(step == tstep)
    def _update():
        lrow = trow - tstep * _BR
        even = (idx % 2) == 0
        rowl = lg_ref[pl.ds(lrow, 1), :]
        tval = jnp.where(even, rowl[0, 0], rowl[0, 1])
        t_ref[...] = jnp.full((1, 128), tval, jnp.float32)

        erow = e2_ref[pl.ds(lrow, 1), :]
        e64 = jnp.where(even, erow[:, :64], erow[:, 64:])
        q = jax.lax.dot_general(e64, wdelta_ref[...], (((1,), (1,)), ((), ())),
                                preferred_element_type=jnp.float32)
        sc = jnp.sum(h2 * (q + bdelta_ref[...]))
        delta = jax.nn.sigmoid(jnp.full((1, 64), sc, jnp.float32))
        u = delta * e64 + (1.0 - delta) * h2
        nrm = jnp.full((1, 64), jnp.sum(u * u), jnp.float32)
        e_new = u * jax.lax.rsqrt(nrm)
        new_row = jnp.where(even,
                            jnp.concatenate([e_new, erow[:, 64:]], axis=1),
                            jnp.concatenate([erow[:, :64], e_new], axis=1))
        eout_ref[pl.ds(lrow, 1), :] = new_row

    @pl.when(step == nsteps - 1)
    def _finish():
        loss_ref[...] = jnp.log(s_ref[...]) + m_ref[...] - t_ref[...]


def kernel(h, r, entity_idx, entity_embeddings, W_ent, b_ent, W_delta, b_delta):
    del r
    m2 = _M // 2
    e2 = entity_embeddings.reshape(m2, 2 * _D)
    idx = jnp.asarray(entity_idx, jnp.int32).reshape(1)
    nsteps = m2 // _BR

    eout, lg2, loss_v = pl.pallas_call(
        _fused_body,
        grid=(nsteps,),
        in_specs=[
            pl.BlockSpec(memory_space=pltpu.SMEM),
            pl.BlockSpec((_BR, 128), lambda i: (i, 0)),
            pl.BlockSpec((1, _D), lambda i: (0, 0)),
            pl.BlockSpec((_D, _D), lambda i: (0, 0)),
            pl.BlockSpec((1, _D), lambda i: (0, 0)),
            pl.BlockSpec((_D, _D), lambda i: (0, 0)),
            pl.BlockSpec((1, _D), lambda i: (0, 0)),
        ],
        out_specs=[
            pl.BlockSpec((_BR, 128), lambda i: (i, 0)),
            pl.BlockSpec((_BR, 2), lambda i: (i, 0)),
            pl.BlockSpec((1, 128), lambda i: (0, 0)),
        ],
        out_shape=[
            jax.ShapeDtypeStruct((m2, 2 * _D), jnp.float32),
            jax.ShapeDtypeStruct((m2, 2), jnp.float32),
            jax.ShapeDtypeStruct((1, 128), jnp.float32),
        ],
        scratch_shapes=[
            pltpu.VMEM((1, 128), jnp.float32),
            pltpu.VMEM((1, 128), jnp.float32),
            pltpu.VMEM((1, 128), jnp.float32),
        ],
    )(idx, e2, h.reshape(1, _D), W_ent, b_ent.reshape(1, _D), W_delta,
      b_delta.reshape(1, _D))

    logits = lg2.reshape(_M)
    loss = loss_v[0, 0]
    e_new = eout.reshape(_M, _D)
    return logits, loss, e_new
